# baseline (speedup 1.0000x reference)
import functools

import jax
import jax.numpy as jnp
from jax import lax
from jax.experimental import pallas as pl
from jax.experimental.pallas import tpu as pltpu
from jax.experimental.pallas import tpu_sc as plsc

N = 10000
E = 320000
D = 128
NC = 2
NS = 16
NW = NC * NS
CH = 80

NPAD = 10240
GPT = NPAD // NW
GCH = GPT // CH

ECH_CH = 80
ECH = 130
ICH = ECH + 2
EPT = ECH * ECH_CH
EPAD = NW * EPT
IPT = ICH * ECH_CH
RPT = NPAD // NS
NSETS = 4

_mesh = plsc.VectorSubcoreMesh(core_axis_name="c", subcore_axis_name="s")


@functools.partial(
    pl.kernel,
    out_type=jax.ShapeDtypeStruct((NPAD, D), jnp.float32),
    mesh=_mesh,
    scratch_types=[
        pltpu.VMEM((CH,), jnp.int32),
        pltpu.VMEM((CH, D), jnp.float32),
        pltpu.SemaphoreType.DMA,
    ],
)
def _sc_gather_h0(emb_hbm, xt_hbm, out_hbm, tidx, rows, sem):
    c = lax.axis_index("c")
    s = lax.axis_index("s")
    base = (c * NS + s) * GPT

    def step(j, carry):
        off = pl.multiple_of(base + j * CH, 8)
        pltpu.sync_copy(xt_hbm.at[pl.ds(off, CH)], tidx)
        pltpu.async_copy(emb_hbm.at[tidx], rows, sem).wait()
        pltpu.sync_copy(rows, out_hbm.at[pl.ds(off, CH)])
        return carry

    lax.fori_loop(0, GCH, step, 0)


def _make_sc_agg():
    idx_t = pltpu.VMEM((ECH_CH,), jnp.int32)
    row_t = pltpu.VMEM((ECH_CH, D), jnp.float32)
    sem_t = pltpu.SemaphoreType.DMA

    @functools.partial(
        pl.kernel,
        out_type=jax.ShapeDtypeStruct((NC, NPAD, D), jnp.float32),
        mesh=_mesh,
        scratch_types=(
            [idx_t] * NSETS + [idx_t] * NSETS + [row_t] * NSETS
            + [pltpu.VMEM_SHARED((NPAD, D), jnp.float32)]
            + [sem_t] * (3 * NSETS)
        ),
    )
    def _sc_agg(h_hbm, src_hbm, dst_hbm, zer_hbm, out_hbm, *sc):
        sidx = sc[0:NSETS]
        didx = sc[NSETS:2 * NSETS]
        rows = sc[2 * NSETS:3 * NSETS]
        acc = sc[3 * NSETS]
        semi = sc[3 * NSETS + 1:3 * NSETS + 1 + NSETS]
        semg = sc[3 * NSETS + 1 + NSETS:3 * NSETS + 1 + 2 * NSETS]
        sems = sc[3 * NSETS + 1 + 2 * NSETS:3 * NSETS + 1 + 3 * NSETS]
        c = lax.axis_index("c")
        s = lax.axis_index("s")
        w = c * NS + s
        pltpu.sync_copy(zer_hbm, acc.at[pl.ds(s * RPT, RPT)])

        def load_idx(chunk, b):
            off = pl.multiple_of(w * IPT + chunk * ECH_CH, 8)
            pltpu.async_copy(src_hbm.at[pl.ds(off, ECH_CH)], sidx[b],
                             semi[b])
            pltpu.async_copy(dst_hbm.at[pl.ds(off, ECH_CH)], didx[b],
                             semi[b])

        def wait_idx(b):
            pltpu.make_async_copy(src_hbm.at[pl.ds(0, ECH_CH)], sidx[b],
                                  semi[b]).wait()
            pltpu.make_async_copy(dst_hbm.at[pl.ds(0, ECH_CH)], didx[b],
                                  semi[b]).wait()

        def gather(b):
            pltpu.async_copy(h_hbm.at[sidx[b]], rows[b], semg[b])

        def wait_gather(b):
            pltpu.make_async_copy(h_hbm.at[sidx[b]], rows[b],
                                  semg[b]).wait()

        def scatter(b):
            pltpu.async_copy(rows[b], acc.at[didx[b]], sems[b], add=True)

        def wait_scatter(b):
            pltpu.make_async_copy(rows[b], acc.at[didx[b]],
                                  sems[b]).wait()

        plsc.subcore_barrier()

        load_idx(0, 0)
        load_idx(1, 1)
        wait_idx(0)
        gather(0)
        load_idx(2, 2)
        wait_idx(1)
        gather(1)
        wait_gather(0)
        scatter(0)
        load_idx(3, 3)
        wait_idx(2)
        gather(2)
        wait_gather(1)
        scatter(1)

        def steady(j, carry):
            a = 2 + j * NSETS
            for k in range(NSETS):
                p = a + k
                bs = k % NSETS
                bg = (3 + k) % NSETS
                bp = (2 + k) % NSETS
                wait_scatter(bs)
                load_idx(p + 2, bs)
                wait_idx(bg)
                gather(bg)
                wait_gather(bp)
                scatter(bp)
            return carry

        lax.fori_loop(0, (ECH - 2) // NSETS, steady, 0)

        wait_idx(3)
        wait_gather(2)
        wait_scatter(0)
        wait_scatter(1)

        plsc.subcore_barrier()
        pltpu.sync_copy(acc.at[pl.ds(s * RPT, RPT)],
                        out_hbm.at[c, pl.ds(s * RPT, RPT)])

    return _sc_agg


_sc_agg_l1 = _make_sc_agg()
_sc_agg_l2 = _make_sc_agg()


def _tc_layer_body(h_ref, p_ref, wa_ref, ba_ref, wb_ref, bb_ref, g_ref,
                   be_ref, out_ref):
    x = h_ref[:N] + p_ref[0, :N] + p_ref[1, :N]
    z = jnp.maximum(
        jnp.dot(x, wa_ref[...], preferred_element_type=jnp.float32)
        + ba_ref[...], 0.0)
    t = (jnp.dot(z, wb_ref[...], preferred_element_type=jnp.float32)
         + bb_ref[...])
    mean = jnp.mean(t, axis=0, keepdims=True)
    var = jnp.mean((t - mean) ** 2, axis=0, keepdims=True)
    out_ref[...] = jnp.maximum(
        g_ref[...] * (t - mean) / jnp.sqrt(var + 1e-5) + be_ref[...], 0.0)


_tc_layer = pl.pallas_call(
    _tc_layer_body,
    out_shape=jax.ShapeDtypeStruct((N, D), jnp.float32),
)


def _tc_layer_out_body(h_ref, p_ref, wa_ref, ba_ref, wb_ref, bb_ref, g_ref,
                       be_ref, wo_ref, bo_ref, out_ref):
    x = h_ref[:N] + p_ref[0, :N] + p_ref[1, :N]
    z = jnp.maximum(
        jnp.dot(x, wa_ref[...], preferred_element_type=jnp.float32)
        + ba_ref[...], 0.0)
    t = (jnp.dot(z, wb_ref[...], preferred_element_type=jnp.float32)
         + bb_ref[...])
    mean = jnp.mean(t, axis=0, keepdims=True)
    var = jnp.mean((t - mean) ** 2, axis=0, keepdims=True)
    r = jnp.maximum(
        g_ref[...] * (t - mean) / jnp.sqrt(var + 1e-5) + be_ref[...], 0.0)
    out_ref[...] = (jnp.dot(r, wo_ref[...], preferred_element_type=jnp.float32)
                    + bo_ref[...])


def _tc_layer_out(a):
    return pl.pallas_call(
        _tc_layer_out_body,
        out_shape=jax.ShapeDtypeStruct((N, a), jnp.float32),
    )


def kernel(x_atom_type, edge_index, batch, emb, W1a, b1a, W1b, b1b, gamma1,
           beta1, W2a, b2a, W2b, b2b, gamma2, beta2, Wout, bout):
    src = edge_index[0]
    dst = edge_index[1]
    src_p = jnp.pad(
        jnp.concatenate([src, jnp.zeros((EPAD - E,), src.dtype)]
                        ).reshape(NW, EPT),
        ((0, 0), (0, IPT - EPT))).reshape(-1)
    dst_p = jnp.pad(
        jnp.concatenate([dst, jnp.full((EPAD - E,), N, dst.dtype)]
                        ).reshape(NW, EPT),
        ((0, 0), (0, IPT - EPT)), constant_values=N).reshape(-1)
    x_pad = jnp.concatenate(
        [x_atom_type.astype(jnp.int32),
         jnp.zeros((NPAD - N,), jnp.int32)])
    zer = jnp.zeros((RPT, D), jnp.float32)

    h0p = _sc_gather_h0(emb, x_pad)
    p1 = _sc_agg_l1(h0p, src_p, dst_p, zer)
    h1 = _tc_layer(h0p, p1, W1a, b1a[None], W1b, b1b[None],
                   gamma1[None], beta1[None])
    p2 = _sc_agg_l2(h1, src_p, dst_p, zer)
    logits = _tc_layer_out(Wout.shape[1])(
        h1, p2, W2a, b2a[None], W2b, b2b[None], gamma2[None],
        beta2[None], Wout, bout[None])
    return logits


# speedup vs baseline: 1.0248x; 1.0248x over previous
import functools

import jax
import jax.numpy as jnp
from jax import lax
from jax.experimental import pallas as pl
from jax.experimental.pallas import tpu as pltpu
from jax.experimental.pallas import tpu_sc as plsc

N = 10000
E = 320000
D = 128
NC = 2
NS = 16
NW = NC * NS
CH = 80

NPAD = 10240
GPT = NPAD // NW
GCH = GPT // CH

ECH_CH = 80
ECH = 130
ICH = 136
EPT = ECH * ECH_CH
EPAD = NW * EPT
IPT = ICH * ECH_CH
RPT = NPAD // NS

_mesh = plsc.VectorSubcoreMesh(core_axis_name="c", subcore_axis_name="s")


@functools.partial(
    pl.kernel,
    out_type=jax.ShapeDtypeStruct((NPAD, D), jnp.float32),
    mesh=_mesh,
    scratch_types=[
        pltpu.VMEM((CH,), jnp.int32),
        pltpu.VMEM((CH, D), jnp.float32),
        pltpu.SemaphoreType.DMA,
    ],
)
def _sc_gather_h0(emb_hbm, xt_hbm, out_hbm, tidx, rows, sem):
    c = lax.axis_index("c")
    s = lax.axis_index("s")
    base = (c * NS + s) * GPT

    def step(j, carry):
        off = pl.multiple_of(base + j * CH, 8)
        pltpu.sync_copy(xt_hbm.at[pl.ds(off, CH)], tidx)
        pltpu.async_copy(emb_hbm.at[tidx], rows, sem).wait()
        pltpu.sync_copy(rows, out_hbm.at[pl.ds(off, CH)])
        return carry

    lax.fori_loop(0, GCH, step, 0)


def _make_sc_agg():
    @functools.partial(
        pl.kernel,
        out_type=jax.ShapeDtypeStruct((NC, NPAD, D), jnp.float32),
        mesh=_mesh,
        scratch_types=[
            pltpu.VMEM((IPT,), jnp.int32),
            pltpu.VMEM((ICH, ECH_CH), jnp.int32),
            pltpu.VMEM((ECH_CH, D), jnp.float32),
            pltpu.VMEM((ECH_CH, D), jnp.float32),
            pltpu.VMEM_SHARED((NPAD, D), jnp.float32),
            pltpu.SemaphoreType.DMA,
            pltpu.SemaphoreType.DMA,
        ],
    )
    def _sc_agg(h_hbm, src_hbm, dst_hbm, zer_hbm, out_hbm, sidx, didx,
                rows0, rows1, acc, sem0, sem1):
        c = lax.axis_index("c")
        s = lax.axis_index("s")
        w = c * NS + s
        pltpu.sync_copy(src_hbm.at[pl.ds(w * IPT, IPT)], sidx)
        pltpu.sync_copy(dst_hbm.at[pl.ds(w * ICH, ICH)], didx)
        pltpu.sync_copy(zer_hbm, acc.at[pl.ds(s * RPT, RPT)])
        plsc.subcore_barrier()

        def gather(p, rows, sem):
            pltpu.async_copy(
                h_hbm.at[sidx.at[pl.ds(p * ECH_CH, ECH_CH)]], rows, sem)

        def wait_scatter(p, rows, sem):
            pltpu.make_async_copy(
                h_hbm.at[sidx.at[pl.ds(0, ECH_CH)]], rows, sem).wait()
            pltpu.sync_copy(rows, acc.at[didx.at[p]], add=True)

        gather(0, rows0, sem0)

        def step(j, carry):
            a = j * 2
            gather(a + 1, rows1, sem1)
            wait_scatter(a, rows0, sem0)
            gather(a + 2, rows0, sem0)
            wait_scatter(a + 1, rows1, sem1)
            return carry

        lax.fori_loop(0, ECH // 2, step, 0)
        pltpu.make_async_copy(
            h_hbm.at[sidx.at[pl.ds(0, ECH_CH)]], rows0, sem0).wait()

        plsc.subcore_barrier()
        pltpu.sync_copy(acc.at[pl.ds(s * RPT, RPT)],
                        out_hbm.at[c, pl.ds(s * RPT, RPT)])

    return _sc_agg


_sc_agg_l1 = _make_sc_agg()
_sc_agg_l2 = _make_sc_agg()


def _tc_layer_body(h_ref, p_ref, wa_ref, ba_ref, wb_ref, bb_ref, g_ref,
                   be_ref, out_ref):
    x = h_ref[:N] + p_ref[0, :N] + p_ref[1, :N]
    z = jnp.maximum(
        jnp.dot(x, wa_ref[...], preferred_element_type=jnp.float32)
        + ba_ref[...], 0.0)
    t = (jnp.dot(z, wb_ref[...], preferred_element_type=jnp.float32)
         + bb_ref[...])
    mean = jnp.mean(t, axis=0, keepdims=True)
    var = jnp.mean((t - mean) ** 2, axis=0, keepdims=True)
    out_ref[...] = jnp.maximum(
        g_ref[...] * (t - mean) / jnp.sqrt(var + 1e-5) + be_ref[...], 0.0)


_tc_layer = pl.pallas_call(
    _tc_layer_body,
    out_shape=jax.ShapeDtypeStruct((N, D), jnp.float32),
)


def _tc_layer_out_body(h_ref, p_ref, wa_ref, ba_ref, wb_ref, bb_ref, g_ref,
                       be_ref, wo_ref, bo_ref, out_ref):
    x = h_ref[:N] + p_ref[0, :N] + p_ref[1, :N]
    z = jnp.maximum(
        jnp.dot(x, wa_ref[...], preferred_element_type=jnp.float32)
        + ba_ref[...], 0.0)
    t = (jnp.dot(z, wb_ref[...], preferred_element_type=jnp.float32)
         + bb_ref[...])
    mean = jnp.mean(t, axis=0, keepdims=True)
    var = jnp.mean((t - mean) ** 2, axis=0, keepdims=True)
    r = jnp.maximum(
        g_ref[...] * (t - mean) / jnp.sqrt(var + 1e-5) + be_ref[...], 0.0)
    out_ref[...] = (jnp.dot(r, wo_ref[...], preferred_element_type=jnp.float32)
                    + bo_ref[...])


def _tc_layer_out(a):
    return pl.pallas_call(
        _tc_layer_out_body,
        out_shape=jax.ShapeDtypeStruct((N, a), jnp.float32),
    )


def kernel(x_atom_type, edge_index, batch, emb, W1a, b1a, W1b, b1b, gamma1,
           beta1, W2a, b2a, W2b, b2b, gamma2, beta2, Wout, bout):
    src = edge_index[0]
    dst = edge_index[1]
    src_p = jnp.pad(
        jnp.concatenate([src, jnp.zeros((EPAD - E,), src.dtype)]
                        ).reshape(NW, EPT),
        ((0, 0), (0, IPT - EPT))).reshape(-1)
    dst_p = jnp.pad(
        jnp.concatenate([dst, jnp.full((EPAD - E,), N, dst.dtype)]
                        ).reshape(NW, EPT),
        ((0, 0), (0, IPT - EPT)),
        constant_values=N).reshape(NW * ICH, ECH_CH)
    x_pad = jnp.concatenate(
        [x_atom_type.astype(jnp.int32),
         jnp.zeros((NPAD - N,), jnp.int32)])
    zer = jnp.zeros((RPT, D), jnp.float32)

    h0p = _sc_gather_h0(emb, x_pad)
    p1 = _sc_agg_l1(h0p, src_p, dst_p, zer)
    h1 = _tc_layer(h0p, p1, W1a, b1a[None], W1b, b1b[None],
                   gamma1[None], beta1[None])
    p2 = _sc_agg_l2(h1, src_p, dst_p, zer)
    logits = _tc_layer_out(Wout.shape[1])(
        h1, p2, W2a, b2a[None], W2b, b2b[None], gamma2[None],
        beta2[None], Wout, bout[None])
    return logits


# speedup vs baseline: 2.3216x; 2.2653x over previous
import functools

import jax
import jax.numpy as jnp
from jax import lax
from jax.experimental import pallas as pl
from jax.experimental.pallas import tpu as pltpu
from jax.experimental.pallas import tpu_sc as plsc

N = 10000
E = 320000
D = 128
NC = 2
NS = 16
NW = NC * NS
CH = 80

NPAD = 10240
GPT = NPAD // NW
GCH = GPT // CH

ECH_CH = 80
ECH = 126
ICH = ECH + 2
EPT = ECH * ECH_CH
EPAD = NW * EPT
IPT = ICH * ECH_CH
RPT = NPAD // NS

_mesh = plsc.VectorSubcoreMesh(core_axis_name="c", subcore_axis_name="s")


@functools.partial(
    pl.kernel,
    out_type=jax.ShapeDtypeStruct((NPAD, D), jnp.float32),
    mesh=_mesh,
    scratch_types=[
        pltpu.VMEM((CH,), jnp.int32),
        pltpu.VMEM((CH, D), jnp.float32),
        pltpu.SemaphoreType.DMA,
    ],
)
def _sc_gather_h0(emb_hbm, xt_hbm, out_hbm, tidx, rows, sem):
    c = lax.axis_index("c")
    s = lax.axis_index("s")
    base = (c * NS + s) * GPT

    def step(j, carry):
        off = pl.multiple_of(base + j * CH, 8)
        pltpu.sync_copy(xt_hbm.at[pl.ds(off, CH)], tidx)
        pltpu.async_copy(emb_hbm.at[tidx], rows, sem).wait()
        pltpu.sync_copy(rows, out_hbm.at[pl.ds(off, CH)])
        return carry

    lax.fori_loop(0, GCH, step, 0)


def _make_sc_agg():
    @functools.partial(
        pl.kernel,
        out_type=jax.ShapeDtypeStruct((NC, NPAD, D), jnp.float32),
        mesh=_mesh,
        scratch_types=[
            pltpu.VMEM((ECH_CH,), jnp.int32),
            pltpu.VMEM((ECH_CH,), jnp.int32),
            pltpu.VMEM((ECH_CH,), jnp.int32),
            pltpu.VMEM((ECH_CH,), jnp.int32),
            pltpu.VMEM((ECH_CH, D), jnp.float32),
            pltpu.VMEM((ECH_CH, D), jnp.float32),
            pltpu.VMEM_SHARED((NPAD, D), jnp.float32),
            pltpu.SemaphoreType.DMA,
            pltpu.SemaphoreType.DMA,
            pltpu.SemaphoreType.DMA,
            pltpu.SemaphoreType.DMA,
        ],
    )
    def _sc_agg(h_hbm, src_hbm, dst_hbm, zer_hbm, out_hbm, sidx0, sidx1,
                didx0, didx1, rows0, rows1, acc, semi0, semi1, semg0, semg1):
        c = lax.axis_index("c")
        s = lax.axis_index("s")
        w = c * NS + s
        pltpu.sync_copy(zer_hbm, acc.at[pl.ds(s * RPT, RPT)])

        bufs = ((sidx0, didx0, rows0, semi0, semg0),
                (sidx1, didx1, rows1, semi1, semg1))

        def load_idx(chunk, b):
            sidx, didx, _, semi, _ = bufs[b]
            off = pl.multiple_of(w * IPT + chunk * ECH_CH, 8)
            pltpu.async_copy(src_hbm.at[pl.ds(off, ECH_CH)], sidx, semi)
            pltpu.async_copy(dst_hbm.at[pl.ds(off, ECH_CH)], didx, semi)

        def wait_idx(b):
            sidx, didx, _, semi, _ = bufs[b]
            pltpu.make_async_copy(src_hbm.at[pl.ds(0, ECH_CH)], sidx,
                                  semi).wait()
            pltpu.make_async_copy(dst_hbm.at[pl.ds(0, ECH_CH)], didx,
                                  semi).wait()

        def gather(b):
            sidx, _, rows, _, semg = bufs[b]
            pltpu.async_copy(h_hbm.at[sidx], rows, semg)

        def wait_scatter(b):
            sidx, didx, rows, _, semg = bufs[b]
            pltpu.make_async_copy(h_hbm.at[sidx], rows, semg).wait()
            pltpu.sync_copy(rows, acc.at[didx], add=True)

        plsc.subcore_barrier()

        load_idx(0, 0)
        load_idx(1, 1)
        wait_idx(0)
        gather(0)

        def step(j, carry):
            a = j * 2
            wait_idx(1)
            gather(1)
            wait_scatter(0)
            load_idx(a + 2, 0)
            wait_idx(0)
            gather(0)
            wait_scatter(1)
            load_idx(a + 3, 1)
            return carry

        lax.fori_loop(0, ECH // 2, step, 0)
        pltpu.make_async_copy(h_hbm.at[sidx0], rows0, semg0).wait()
        wait_idx(1)

        plsc.subcore_barrier()
        pltpu.sync_copy(acc.at[pl.ds(s * RPT, RPT)],
                        out_hbm.at[c, pl.ds(s * RPT, RPT)])

    return _sc_agg


_sc_agg_l1 = _make_sc_agg()
_sc_agg_l2 = _make_sc_agg()


def _tc_layer_body(h_ref, p_ref, wa_ref, ba_ref, wb_ref, bb_ref, g_ref,
                   be_ref, out_ref):
    x = h_ref[:N] + p_ref[0, :N] + p_ref[1, :N]
    z = jnp.maximum(
        jnp.dot(x, wa_ref[...], preferred_element_type=jnp.float32)
        + ba_ref[...], 0.0)
    t = (jnp.dot(z, wb_ref[...], preferred_element_type=jnp.float32)
         + bb_ref[...])
    mean = jnp.mean(t, axis=0, keepdims=True)
    var = jnp.mean((t - mean) ** 2, axis=0, keepdims=True)
    out_ref[...] = jnp.maximum(
        g_ref[...] * (t - mean) / jnp.sqrt(var + 1e-5) + be_ref[...], 0.0)


_tc_layer = pl.pallas_call(
    _tc_layer_body,
    out_shape=jax.ShapeDtypeStruct((N, D), jnp.float32),
)


def _tc_layer_out_body(h_ref, p_ref, wa_ref, ba_ref, wb_ref, bb_ref, g_ref,
                       be_ref, wo_ref, bo_ref, out_ref):
    x = h_ref[:N] + p_ref[0, :N] + p_ref[1, :N]
    z = jnp.maximum(
        jnp.dot(x, wa_ref[...], preferred_element_type=jnp.float32)
        + ba_ref[...], 0.0)
    t = (jnp.dot(z, wb_ref[...], preferred_element_type=jnp.float32)
         + bb_ref[...])
    mean = jnp.mean(t, axis=0, keepdims=True)
    var = jnp.mean((t - mean) ** 2, axis=0, keepdims=True)
    r = jnp.maximum(
        g_ref[...] * (t - mean) / jnp.sqrt(var + 1e-5) + be_ref[...], 0.0)
    out_ref[...] = (jnp.dot(r, wo_ref[...], preferred_element_type=jnp.float32)
                    + bo_ref[...])


def _tc_layer_out(a):
    return pl.pallas_call(
        _tc_layer_out_body,
        out_shape=jax.ShapeDtypeStruct((N, a), jnp.float32),
    )


def kernel(x_atom_type, edge_index, batch, emb, W1a, b1a, W1b, b1b, gamma1,
           beta1, W2a, b2a, W2b, b2b, gamma2, beta2, Wout, bout):
    src = edge_index[0]
    dst = edge_index[1]
    src_p = jnp.pad(
        jnp.concatenate([src, jnp.zeros((EPAD - E,), src.dtype)]
                        ).reshape(NW, EPT),
        ((0, 0), (0, IPT - EPT))).reshape(-1)
    dst_p = jnp.pad(
        jnp.concatenate([dst, jnp.full((EPAD - E,), N, dst.dtype)]
                        ).reshape(NW, EPT),
        ((0, 0), (0, IPT - EPT)), constant_values=N).reshape(-1)
    x_pad = jnp.concatenate(
        [x_atom_type.astype(jnp.int32),
         jnp.zeros((NPAD - N,), jnp.int32)])
    zer = jnp.zeros((RPT, D), jnp.float32)

    h0p = _sc_gather_h0(emb, x_pad)
    p1 = _sc_agg_l1(h0p, src_p, dst_p, zer)
    h1 = _tc_layer(h0p, p1, W1a, b1a[None], W1b, b1b[None],
                   gamma1[None], beta1[None])
    p2 = _sc_agg_l2(h1, src_p, dst_p, zer)
    logits = _tc_layer_out(Wout.shape[1])(
        h1, p2, W2a, b2a[None], W2b, b2b[None], gamma2[None],
        beta2[None], Wout, bout[None])
    return logits


# speedup vs baseline: 3.3445x; 1.4406x over previous
import functools

import jax
import jax.numpy as jnp
from jax import lax
from jax.experimental import pallas as pl
from jax.experimental.pallas import tpu as pltpu
from jax.experimental.pallas import tpu_sc as plsc

N = 10000
E = 320000
D = 128
NC = 2
NS = 16
NW = NC * NS
CH = 80

NPAD = 10240
GPT = NPAD // NW
GCH = GPT // CH

ECH_CH = 80
ECH = 126
ICH = ECH
EPT = ECH * ECH_CH
EPAD = NW * EPT
IPT = ICH * ECH_CH
RPT = NPAD // NS

_mesh = plsc.VectorSubcoreMesh(core_axis_name="c", subcore_axis_name="s")


@functools.partial(
    pl.kernel,
    out_type=jax.ShapeDtypeStruct((NPAD, D), jnp.float32),
    mesh=_mesh,
    scratch_types=[
        pltpu.VMEM((CH,), jnp.int32),
        pltpu.VMEM((CH, D), jnp.float32),
        pltpu.SemaphoreType.DMA,
    ],
)
def _sc_gather_h0(emb_hbm, xt_hbm, out_hbm, tidx, rows, sem):
    c = lax.axis_index("c")
    s = lax.axis_index("s")
    base = (c * NS + s) * GPT

    def step(j, carry):
        off = pl.multiple_of(base + j * CH, 8)
        pltpu.sync_copy(xt_hbm.at[pl.ds(off, CH)], tidx)
        pltpu.async_copy(emb_hbm.at[tidx], rows, sem).wait()
        pltpu.sync_copy(rows, out_hbm.at[pl.ds(off, CH)])
        return carry

    lax.fori_loop(0, GCH, step, 0)


def _make_sc_agg():
    @functools.partial(
        pl.kernel,
        out_type=jax.ShapeDtypeStruct((NC, NPAD, D), jnp.float32),
        mesh=_mesh,
        scratch_types=[
            pltpu.VMEM((ECH_CH,), jnp.int32),
            pltpu.VMEM((ECH_CH,), jnp.int32),
            pltpu.VMEM((ECH_CH,), jnp.int32),
            pltpu.VMEM((ECH_CH,), jnp.int32),
            pltpu.VMEM((ECH_CH, D), jnp.float32),
            pltpu.VMEM((ECH_CH, D), jnp.float32),
            pltpu.VMEM_SHARED((NPAD, D), jnp.float32),
            pltpu.SemaphoreType.DMA,
            pltpu.SemaphoreType.DMA,
            pltpu.SemaphoreType.DMA,
            pltpu.SemaphoreType.DMA,
        ],
    )
    def _sc_agg(h_hbm, src_hbm, dst_hbm, zer_hbm, out_hbm, sidx0, sidx1,
                didx0, didx1, rows0, rows1, acc, semi0, semi1, semg0, semg1):
        c = lax.axis_index("c")
        s = lax.axis_index("s")
        w = c * NS + s
        pltpu.sync_copy(zer_hbm, acc.at[pl.ds(s * RPT, RPT)])

        bufs = ((sidx0, didx0, rows0, semi0, semg0),
                (sidx1, didx1, rows1, semi1, semg1))

        def load_idx(chunk, b):
            sidx, didx, _, semi, _ = bufs[b]
            off = pl.multiple_of(w * IPT + chunk * ECH_CH, 8)
            pltpu.async_copy(src_hbm.at[pl.ds(off, ECH_CH)], sidx, semi)
            pltpu.async_copy(dst_hbm.at[pl.ds(off, ECH_CH)], didx, semi)

        def wait_idx(b):
            sidx, didx, _, semi, _ = bufs[b]
            pltpu.make_async_copy(src_hbm.at[pl.ds(0, ECH_CH)], sidx,
                                  semi).wait()
            pltpu.make_async_copy(dst_hbm.at[pl.ds(0, ECH_CH)], didx,
                                  semi).wait()

        def gather(b):
            sidx, _, rows, _, semg = bufs[b]
            pltpu.async_copy(h_hbm.at[sidx], rows, semg)

        def wait_scatter(b):
            sidx, didx, rows, _, semg = bufs[b]
            pltpu.make_async_copy(h_hbm.at[sidx], rows, semg).wait()
            pltpu.sync_copy(rows, acc.at[didx], add=True)

        plsc.subcore_barrier()

        load_idx(0, 0)
        load_idx(1, 1)
        wait_idx(0)
        gather(0)

        def step(j, carry):
            a = j * 2
            wait_idx(1)
            gather(1)
            wait_scatter(0)

            @pl.when(a + 2 < ECH)
            def _():
                load_idx(a + 2, 0)
                wait_idx(0)
                gather(0)

            wait_scatter(1)

            @pl.when(a + 3 < ECH)
            def _():
                load_idx(a + 3, 1)

            return carry

        lax.fori_loop(0, ECH // 2, step, 0)

        plsc.subcore_barrier()
        pltpu.sync_copy(acc.at[pl.ds(s * RPT, RPT)],
                        out_hbm.at[c, pl.ds(s * RPT, RPT)])

    return _sc_agg


_sc_agg_l1 = _make_sc_agg()
_sc_agg_l2 = _make_sc_agg()


def _tc_layer_body(h_ref, p_ref, wa_ref, ba_ref, wb_ref, bb_ref, g_ref,
                   be_ref, out_ref):
    x = h_ref[:N] + p_ref[0, :N] + p_ref[1, :N]
    z = jnp.maximum(
        jnp.dot(x, wa_ref[...], preferred_element_type=jnp.float32)
        + ba_ref[...], 0.0)
    t = (jnp.dot(z, wb_ref[...], preferred_element_type=jnp.float32)
         + bb_ref[...])
    mean = jnp.mean(t, axis=0, keepdims=True)
    var = jnp.mean((t - mean) ** 2, axis=0, keepdims=True)
    out_ref[...] = jnp.maximum(
        g_ref[...] * (t - mean) / jnp.sqrt(var + 1e-5) + be_ref[...], 0.0)


_tc_layer = pl.pallas_call(
    _tc_layer_body,
    out_shape=jax.ShapeDtypeStruct((N, D), jnp.float32),
)


def _tc_layer_out_body(h_ref, p_ref, wa_ref, ba_ref, wb_ref, bb_ref, g_ref,
                       be_ref, wo_ref, bo_ref, out_ref):
    x = h_ref[:N] + p_ref[0, :N] + p_ref[1, :N]
    z = jnp.maximum(
        jnp.dot(x, wa_ref[...], preferred_element_type=jnp.float32)
        + ba_ref[...], 0.0)
    t = (jnp.dot(z, wb_ref[...], preferred_element_type=jnp.float32)
         + bb_ref[...])
    mean = jnp.mean(t, axis=0, keepdims=True)
    var = jnp.mean((t - mean) ** 2, axis=0, keepdims=True)
    r = jnp.maximum(
        g_ref[...] * (t - mean) / jnp.sqrt(var + 1e-5) + be_ref[...], 0.0)
    out_ref[...] = (jnp.dot(r, wo_ref[...], preferred_element_type=jnp.float32)
                    + bo_ref[...])


def _tc_layer_out(a):
    return pl.pallas_call(
        _tc_layer_out_body,
        out_shape=jax.ShapeDtypeStruct((N, a), jnp.float32),
    )


def kernel(x_atom_type, edge_index, batch, emb, W1a, b1a, W1b, b1b, gamma1,
           beta1, W2a, b2a, W2b, b2b, gamma2, beta2, Wout, bout):
    src = edge_index[0]
    dst = edge_index[1]
    src_p = jnp.pad(
        jnp.concatenate([src, jnp.zeros((EPAD - E,), src.dtype)]
                        ).reshape(NW, EPT),
        ((0, 0), (0, IPT - EPT))).reshape(-1)
    dst_p = jnp.pad(
        jnp.concatenate([dst, jnp.full((EPAD - E,), N, dst.dtype)]
                        ).reshape(NW, EPT),
        ((0, 0), (0, IPT - EPT)), constant_values=N).reshape(-1)
    x_pad = jnp.concatenate(
        [x_atom_type.astype(jnp.int32),
         jnp.zeros((NPAD - N,), jnp.int32)])
    zer = jnp.zeros((RPT, D), jnp.float32)

    h0p = _sc_gather_h0(emb, x_pad)
    p1 = _sc_agg_l1(h0p, src_p, dst_p, zer)
    h1 = _tc_layer(h0p, p1, W1a, b1a[None], W1b, b1b[None],
                   gamma1[None], beta1[None])
    p2 = _sc_agg_l2(h1, src_p, dst_p, zer)
    logits = _tc_layer_out(Wout.shape[1])(
        h1, p2, W2a, b2a[None], W2b, b2b[None], gamma2[None],
        beta2[None], Wout, bout[None])
    return logits


# speedup vs baseline: 3.5863x; 1.0723x over previous
import functools

import jax
import jax.numpy as jnp
from jax import lax
from jax.experimental import pallas as pl
from jax.experimental.pallas import tpu as pltpu
from jax.experimental.pallas import tpu_sc as plsc

N = 10000
E = 320000
D = 128
NC = 2
NS = 16
NW = NC * NS
CH = 80

NPAD = 10240
GPT = NPAD // NW
GCH = GPT // CH

ECH_CH = 80
ECH = 126
ICH = ECH
EPT = ECH * ECH_CH
EPAD = NW * EPT
IPT = ICH * ECH_CH
RPT = NPAD // NS

_mesh = plsc.VectorSubcoreMesh(core_axis_name="c", subcore_axis_name="s")


@functools.partial(
    pl.kernel,
    out_type=jax.ShapeDtypeStruct((NPAD, D), jnp.float32),
    mesh=_mesh,
    scratch_types=[
        pltpu.VMEM((CH,), jnp.int32),
        pltpu.VMEM((CH, D), jnp.float32),
        pltpu.SemaphoreType.DMA,
    ],
)
def _sc_gather_h0(emb_hbm, xt_hbm, out_hbm, tidx, rows, sem):
    c = lax.axis_index("c")
    s = lax.axis_index("s")
    base = (c * NS + s) * GPT

    def step(j, carry):
        off = pl.multiple_of(base + j * CH, 8)
        pltpu.sync_copy(xt_hbm.at[pl.ds(off, CH)], tidx)
        pltpu.async_copy(emb_hbm.at[tidx], rows, sem).wait()
        pltpu.sync_copy(rows, out_hbm.at[pl.ds(off, CH)])
        return carry

    lax.fori_loop(0, GCH, step, 0)


def _make_sc_agg():
    @functools.partial(
        pl.kernel,
        out_type=jax.ShapeDtypeStruct((NC, NPAD, D), jnp.float32),
        mesh=_mesh,
        scratch_types=[
            pltpu.VMEM((ECH_CH,), jnp.int32),
            pltpu.VMEM((ECH_CH,), jnp.int32),
            pltpu.VMEM((ECH_CH,), jnp.int32),
            pltpu.VMEM((ECH_CH,), jnp.int32),
            pltpu.VMEM((ECH_CH,), jnp.int32),
            pltpu.VMEM((ECH_CH,), jnp.int32),
            pltpu.VMEM((ECH_CH, D), jnp.float32),
            pltpu.VMEM((ECH_CH, D), jnp.float32),
            pltpu.VMEM_SHARED((NPAD, D), jnp.float32),
            pltpu.SemaphoreType.DMA,
            pltpu.SemaphoreType.DMA,
            pltpu.SemaphoreType.DMA,
            pltpu.SemaphoreType.DMA,
            pltpu.SemaphoreType.DMA,
        ],
    )
    def _sc_agg(h_hbm, src_hbm, dst_hbm, zer_hbm, out_hbm, sidx0, sidx1,
                sidx2, didx0, didx1, didx2, rows0, rows1, acc,
                semi0, semi1, semi2, semg0, semg1):
        c = lax.axis_index("c")
        s = lax.axis_index("s")
        w = c * NS + s
        pltpu.sync_copy(zer_hbm, acc.at[pl.ds(s * RPT, RPT)])

        sidx = (sidx0, sidx1, sidx2)
        didx = (didx0, didx1, didx2)
        semi = (semi0, semi1, semi2)
        rows = (rows0, rows1)
        semg = (semg0, semg1)

        def load_idx(chunk, bi):
            off = pl.multiple_of(w * IPT + chunk * ECH_CH, 8)
            pltpu.async_copy(src_hbm.at[pl.ds(off, ECH_CH)], sidx[bi],
                             semi[bi])
            pltpu.async_copy(dst_hbm.at[pl.ds(off, ECH_CH)], didx[bi],
                             semi[bi])

        def wait_idx(bi):
            pltpu.make_async_copy(src_hbm.at[pl.ds(0, ECH_CH)], sidx[bi],
                                  semi[bi]).wait()
            pltpu.make_async_copy(dst_hbm.at[pl.ds(0, ECH_CH)], didx[bi],
                                  semi[bi]).wait()

        def gather(bi, br):
            pltpu.async_copy(h_hbm.at[sidx[bi]], rows[br], semg[br])

        def wait_scatter(bi, br):
            pltpu.make_async_copy(h_hbm.at[sidx[bi]], rows[br],
                                  semg[br]).wait()
            pltpu.sync_copy(rows[br], acc.at[didx[bi]], add=True)

        plsc.subcore_barrier()

        load_idx(0, 0)
        load_idx(1, 1)
        wait_idx(0)
        gather(0, 0)

        def step(j, carry):
            a = j * 6
            for k in range(6):
                p = a + k

                @pl.when(p + 1 < ECH)
                def _():
                    wait_idx((k + 1) % 3)
                    gather((k + 1) % 3, (k + 1) % 2)

                @pl.when(p + 2 < ECH)
                def _():
                    load_idx(p + 2, (k + 2) % 3)

                wait_scatter(k % 3, k % 2)
            return carry

        lax.fori_loop(0, ECH // 6, step, 0)

        plsc.subcore_barrier()
        pltpu.sync_copy(acc.at[pl.ds(s * RPT, RPT)],
                        out_hbm.at[c, pl.ds(s * RPT, RPT)])

    return _sc_agg


_sc_agg_l1 = _make_sc_agg()
_sc_agg_l2 = _make_sc_agg()


def _tc_layer_body(h_ref, p_ref, wa_ref, ba_ref, wb_ref, bb_ref, g_ref,
                   be_ref, out_ref):
    x = h_ref[:N] + p_ref[0, :N] + p_ref[1, :N]
    z = jnp.maximum(
        jnp.dot(x, wa_ref[...], preferred_element_type=jnp.float32)
        + ba_ref[...], 0.0)
    t = (jnp.dot(z, wb_ref[...], preferred_element_type=jnp.float32)
         + bb_ref[...])
    mean = jnp.mean(t, axis=0, keepdims=True)
    var = jnp.mean((t - mean) ** 2, axis=0, keepdims=True)
    out_ref[...] = jnp.maximum(
        g_ref[...] * (t - mean) / jnp.sqrt(var + 1e-5) + be_ref[...], 0.0)


_tc_layer = pl.pallas_call(
    _tc_layer_body,
    out_shape=jax.ShapeDtypeStruct((N, D), jnp.float32),
)


def _tc_layer_out_body(h_ref, p_ref, wa_ref, ba_ref, wb_ref, bb_ref, g_ref,
                       be_ref, wo_ref, bo_ref, out_ref):
    x = h_ref[:N] + p_ref[0, :N] + p_ref[1, :N]
    z = jnp.maximum(
        jnp.dot(x, wa_ref[...], preferred_element_type=jnp.float32)
        + ba_ref[...], 0.0)
    t = (jnp.dot(z, wb_ref[...], preferred_element_type=jnp.float32)
         + bb_ref[...])
    mean = jnp.mean(t, axis=0, keepdims=True)
    var = jnp.mean((t - mean) ** 2, axis=0, keepdims=True)
    r = jnp.maximum(
        g_ref[...] * (t - mean) / jnp.sqrt(var + 1e-5) + be_ref[...], 0.0)
    out_ref[...] = (jnp.dot(r, wo_ref[...], preferred_element_type=jnp.float32)
                    + bo_ref[...])


def _tc_layer_out(a):
    return pl.pallas_call(
        _tc_layer_out_body,
        out_shape=jax.ShapeDtypeStruct((N, a), jnp.float32),
    )


def kernel(x_atom_type, edge_index, batch, emb, W1a, b1a, W1b, b1b, gamma1,
           beta1, W2a, b2a, W2b, b2b, gamma2, beta2, Wout, bout):
    src = edge_index[0]
    dst = edge_index[1]
    src_p = jnp.pad(
        jnp.concatenate([src, jnp.zeros((EPAD - E,), src.dtype)]
                        ).reshape(NW, EPT),
        ((0, 0), (0, IPT - EPT))).reshape(-1)
    dst_p = jnp.pad(
        jnp.concatenate([dst, jnp.full((EPAD - E,), N, dst.dtype)]
                        ).reshape(NW, EPT),
        ((0, 0), (0, IPT - EPT)), constant_values=N).reshape(-1)
    x_pad = jnp.concatenate(
        [x_atom_type.astype(jnp.int32),
         jnp.zeros((NPAD - N,), jnp.int32)])
    zer = jnp.zeros((RPT, D), jnp.float32)

    h0p = _sc_gather_h0(emb, x_pad)
    p1 = _sc_agg_l1(h0p, src_p, dst_p, zer)
    h1 = _tc_layer(h0p, p1, W1a, b1a[None], W1b, b1b[None],
                   gamma1[None], beta1[None])
    p2 = _sc_agg_l2(h1, src_p, dst_p, zer)
    logits = _tc_layer_out(Wout.shape[1])(
        h1, p2, W2a, b2a[None], W2b, b2b[None], gamma2[None],
        beta2[None], Wout, bout[None])
    return logits
